# baseline (speedup 1.0000x reference)
import functools
import math

import jax
import jax.numpy as jnp
from jax import lax
from jax.experimental import pallas as pl
from jax.experimental.pallas import tpu as pltpu
from jax.experimental.pallas import tpu_sc as plsc

_MBLK = 512
_NCH = 4


def _dist_argmin_kernel(x_ref, cb_ref, idx_ref, esq_ref):
    cb = cb_ref[...]

    @pl.when(pl.program_id(0) == 0)
    def _():
        esq_ref[...] = jnp.sum(cb * cb, axis=1)[None, :]

    x = x_ref[...]
    x_sq = jnp.sum(x * x, axis=1, keepdims=True)
    e_sq = esq_ref[...]
    dot = lax.dot_general(x, cb, (((1,), (1,)), ((), ())),
                          preferred_element_type=jnp.float32)
    dist = x_sq + e_sq - 2.0 * dot
    idx_ref[0, 0, :] = jnp.argmin(dist, axis=1).astype(jnp.int32)


@functools.lru_cache(maxsize=None)
def _make_tc_scorer(n, c, k, blk0, nblk):
    return pl.pallas_call(
        _dist_argmin_kernel,
        grid=(nblk,),
        in_specs=[
            pl.BlockSpec((_MBLK, c), lambda i: (blk0 + i, 0)),
            pl.BlockSpec((k, c), lambda i: (0, 0)),
        ],
        out_specs=pl.BlockSpec((1, 1, _MBLK), lambda i: (i, 0, 0)),
        out_shape=jax.ShapeDtypeStruct((nblk, 1, _MBLK), jnp.int32),
        scratch_shapes=[pltpu.VMEM((1, k), jnp.float32)],
    )


def _dist_argmin_quant_kernel(x_ref, cb_ref, buf_ref, idx_ref, out_ref,
                              esq_ref, cbhi_ref, cblo_ref):
    mblk = x_ref.shape[0]
    kk = cb_ref.shape[0]
    cb = cb_ref[...]

    @pl.when(pl.program_id(0) == 0)
    def _():
        esq_ref[...] = jnp.sum(cb * cb, axis=1)[None, :]
        hi = cb.astype(jnp.bfloat16)
        cbhi_ref[...] = hi
        cblo_ref[...] = (cb - hi.astype(jnp.float32)).astype(jnp.bfloat16)

    x = x_ref[...]
    x_sq = jnp.sum(x * x, axis=1, keepdims=True)
    e_sq = esq_ref[...]
    dot = lax.dot_general(x, cb, (((1,), (1,)), ((), ())),
                          preferred_element_type=jnp.float32)
    dist = x_sq + e_sq - 2.0 * dot
    idx = jnp.argmin(dist, axis=1).astype(jnp.int32)
    idx_ref[0, 0, :] = idx
    iota = lax.broadcasted_iota(jnp.int32, (mblk, kk), 1)
    onehot = (iota == idx[:, None]).astype(jnp.bfloat16)
    q = lax.dot_general(onehot, cbhi_ref[...], (((1,), (0,)), ((), ())),
                        preferred_element_type=jnp.float32)
    q = q + lax.dot_general(onehot, cblo_ref[...], (((1,), (0,)), ((), ())),
                            preferred_element_type=jnp.float32)
    out_ref[...] = q


@functools.lru_cache(maxsize=None)
def _make_tc_scorer_quant(n, c, k, blk0, nblk):
    return pl.pallas_call(
        _dist_argmin_quant_kernel,
        grid=(nblk,),
        in_specs=[
            pl.BlockSpec((_MBLK, c), lambda i: (blk0 + i, 0)),
            pl.BlockSpec((k, c), lambda i: (0, 0)),
            pl.BlockSpec(memory_space=pl.ANY),
        ],
        out_specs=[
            pl.BlockSpec((1, 1, _MBLK), lambda i: (i, 0, 0)),
            pl.BlockSpec((_MBLK, c), lambda i: (blk0 + i, 0)),
        ],
        out_shape=[
            jax.ShapeDtypeStruct((nblk, 1, _MBLK), jnp.int32),
            jax.ShapeDtypeStruct((n, c), jnp.float32),
        ],
        scratch_shapes=[pltpu.VMEM((1, k), jnp.float32),
                        pltpu.VMEM((k, c), jnp.bfloat16),
                        pltpu.VMEM((k, c), jnp.bfloat16)],
        input_output_aliases={2: 1},
    )


@functools.lru_cache(maxsize=None)
def _make_sc_gather(nidx, nout, k, d):
    nc, ns = 2, 16
    nw = nc * ns
    assert nidx % nw == 0
    b_per_w = nidx // nw
    chunk = 96 if b_per_w % 96 == 0 else 72
    assert b_per_w % chunk == 0 and chunk % 8 == 0
    nchunk = b_per_w // chunk
    mesh = plsc.VectorSubcoreMesh(core_axis_name="c", subcore_axis_name="s")

    @functools.partial(
        pl.kernel,
        mesh=mesh,
        out_type=jax.ShapeDtypeStruct((nout, d), jnp.float32),
        scratch_types=[
            pltpu.VMEM((nchunk, chunk), jnp.int32),
            pltpu.VMEM((2, chunk, d), jnp.float32),
            pltpu.SemaphoreType.DMA,
            pltpu.SemaphoreType.DMA,
            pltpu.SemaphoreType.DMA,
            pltpu.SemaphoreType.DMA,
            pltpu.SemaphoreType.DMA,
        ],
    )
    def gather_kernel(cb_hbm, idx_hbm, out_hbm, idx2d, rows_v,
                      isem, g0, g1, w0, w1):
        wid = lax.axis_index("s") * nc + lax.axis_index("c")
        base = wid * b_per_w
        gsem = (g0, g1)
        wsem = (w0, w1)
        cps = [pltpu.async_copy(idx_hbm.at[pl.ds(base + ci * chunk, chunk)],
                                idx2d.at[ci], isem)
               for ci in range(nchunk)]
        for cp in cps:
            cp.wait()
        gathers = [None] * nchunk
        pending_w = [None, None]
        gathers[0] = pltpu.async_copy(cb_hbm.at[idx2d.at[0]],
                                      rows_v.at[0], gsem[0])
        for ci in range(nchunk):
            b = ci % 2
            nb = (ci + 1) % 2
            if ci + 1 < nchunk:
                if pending_w[nb] is not None:
                    pending_w[nb].wait()
                    pending_w[nb] = None
                gathers[ci + 1] = pltpu.async_copy(
                    cb_hbm.at[idx2d.at[ci + 1]], rows_v.at[nb], gsem[nb])
            gathers[ci].wait()
            pending_w[b] = pltpu.async_copy(
                rows_v.at[b], out_hbm.at[pl.ds(base + ci * chunk, chunk)],
                wsem[b])
        for b in range(2):
            if pending_w[b] is not None:
                pending_w[b].wait()

    return gather_kernel


def _copy_block_kernel(q_ref, buf_ref, o_ref):
    o_ref[...] = q_ref[...]


@functools.lru_cache(maxsize=None)
def _make_assembler(n, ch, d, row0):
    nblkc = ch // _MBLK
    blk0 = row0 // _MBLK
    return pl.pallas_call(
        _copy_block_kernel,
        grid=(nblkc,),
        in_specs=[
            pl.BlockSpec((_MBLK, d), lambda i: (i, 0)),
            pl.BlockSpec(memory_space=pl.ANY),
        ],
        out_specs=pl.BlockSpec((_MBLK, d), lambda i: (blk0 + i, 0)),
        out_shape=jax.ShapeDtypeStruct((n, d), jnp.float32),
        input_output_aliases={1: 0},
    )


def kernel(in_feas, codebook):
    bq, lq, cq = in_feas.shape
    x = in_feas.reshape(-1, cq)
    n = x.shape[0]
    k, d = codebook.shape
    nblk = n // _MBLK
    blk_per_ch = nblk // _NCH
    ch = blk_per_ch * _MBLK
    n_sc = _NCH // 2
    idxs = []
    qparts = []
    for i in range(n_sc):
        scorer = _make_tc_scorer(n, cq, k, i * blk_per_ch, blk_per_ch)
        idx_i = scorer(x, codebook).reshape(-1)
        idxs.append(idx_i)
        nout = n if i == 0 else ch
        qparts.append(_make_sc_gather(ch, nout, k, d)(codebook, idx_i))
    quant = qparts[0]
    for i in range(n_sc, _NCH):
        scorer_q = _make_tc_scorer_quant(n, cq, k, i * blk_per_ch, blk_per_ch)
        idx_i, quant = scorer_q(x, codebook, quant)
        idxs.append(idx_i.reshape(-1))
    for i in range(1, n_sc):
        quant = lax.dynamic_update_slice(quant, qparts[i], (i * ch, 0))
    idx = jnp.concatenate(idxs, axis=0)
    h = int(math.sqrt(lq))
    w = lq // h
    return quant.reshape(bq, lq, cq), idx.reshape(bq, h, w)


# speedup vs baseline: 1.0923x; 1.0923x over previous
import functools
import math

import jax
import jax.numpy as jnp
from jax import lax
from jax.experimental import pallas as pl
from jax.experimental.pallas import tpu as pltpu
from jax.experimental.pallas import tpu_sc as plsc

_MBLK = 768
_NCH = 4


def _dist_argmin_kernel(x_ref, cb_ref, idx_ref, esq_ref):
    cb = cb_ref[...]

    @pl.when(pl.program_id(0) == 0)
    def _():
        esq_ref[...] = jnp.sum(cb * cb, axis=1)[None, :]

    x = x_ref[...]
    x_sq = jnp.sum(x * x, axis=1, keepdims=True)
    e_sq = esq_ref[...]
    dot = lax.dot_general(x, cb, (((1,), (1,)), ((), ())),
                          preferred_element_type=jnp.float32)
    dist = x_sq + e_sq - 2.0 * dot
    idx_ref[0, 0, :] = jnp.argmin(dist, axis=1).astype(jnp.int32)


@functools.lru_cache(maxsize=None)
def _make_tc_scorer(n, c, k, blk0, nblk):
    return pl.pallas_call(
        _dist_argmin_kernel,
        grid=(nblk,),
        in_specs=[
            pl.BlockSpec((_MBLK, c), lambda i: (blk0 + i, 0)),
            pl.BlockSpec((k, c), lambda i: (0, 0)),
        ],
        out_specs=pl.BlockSpec((1, 1, _MBLK), lambda i: (i, 0, 0)),
        out_shape=jax.ShapeDtypeStruct((nblk, 1, _MBLK), jnp.int32),
        scratch_shapes=[pltpu.VMEM((1, k), jnp.float32)],
    )


def _dist_argmin_quant_kernel(x_ref, cb_ref, buf_ref, idx_ref, out_ref,
                              esq_ref, cbhi_ref, cblo_ref):
    mblk = x_ref.shape[0]
    kk = cb_ref.shape[0]
    cb = cb_ref[...]

    @pl.when(pl.program_id(0) == 0)
    def _():
        esq_ref[...] = jnp.sum(cb * cb, axis=1)[None, :]
        hi = cb.astype(jnp.bfloat16)
        cbhi_ref[...] = hi
        cblo_ref[...] = (cb - hi.astype(jnp.float32)).astype(jnp.bfloat16)

    x = x_ref[...]
    x_sq = jnp.sum(x * x, axis=1, keepdims=True)
    e_sq = esq_ref[...]
    dot = lax.dot_general(x, cb, (((1,), (1,)), ((), ())),
                          preferred_element_type=jnp.float32)
    dist = x_sq + e_sq - 2.0 * dot
    idx = jnp.argmin(dist, axis=1).astype(jnp.int32)
    idx_ref[0, 0, :] = idx
    iota = lax.broadcasted_iota(jnp.int32, (mblk, kk), 1)
    onehot = (iota == idx[:, None]).astype(jnp.bfloat16)
    q = lax.dot_general(onehot, cbhi_ref[...], (((1,), (0,)), ((), ())),
                        preferred_element_type=jnp.float32)
    q = q + lax.dot_general(onehot, cblo_ref[...], (((1,), (0,)), ((), ())),
                            preferred_element_type=jnp.float32)
    out_ref[...] = q


@functools.lru_cache(maxsize=None)
def _make_tc_scorer_quant(n, c, k, blk0, nblk):
    return pl.pallas_call(
        _dist_argmin_quant_kernel,
        grid=(nblk,),
        in_specs=[
            pl.BlockSpec((_MBLK, c), lambda i: (blk0 + i, 0)),
            pl.BlockSpec((k, c), lambda i: (0, 0)),
            pl.BlockSpec(memory_space=pl.ANY),
        ],
        out_specs=[
            pl.BlockSpec((1, 1, _MBLK), lambda i: (i, 0, 0)),
            pl.BlockSpec((_MBLK, c), lambda i: (blk0 + i, 0)),
        ],
        out_shape=[
            jax.ShapeDtypeStruct((nblk, 1, _MBLK), jnp.int32),
            jax.ShapeDtypeStruct((n, c), jnp.float32),
        ],
        scratch_shapes=[pltpu.VMEM((1, k), jnp.float32),
                        pltpu.VMEM((k, c), jnp.bfloat16),
                        pltpu.VMEM((k, c), jnp.bfloat16)],
        input_output_aliases={2: 1},
    )


@functools.lru_cache(maxsize=None)
def _make_sc_gather(nidx, nout, k, d):
    nc, ns = 2, 16
    nw = nc * ns
    assert nidx % nw == 0
    b_per_w = nidx // nw
    chunk = 96 if b_per_w % 96 == 0 else 72
    assert b_per_w % chunk == 0 and chunk % 8 == 0
    nchunk = b_per_w // chunk
    mesh = plsc.VectorSubcoreMesh(core_axis_name="c", subcore_axis_name="s")

    @functools.partial(
        pl.kernel,
        mesh=mesh,
        out_type=jax.ShapeDtypeStruct((nout, d), jnp.float32),
        scratch_types=[
            pltpu.VMEM((nchunk, chunk), jnp.int32),
            pltpu.VMEM((2, chunk, d), jnp.float32),
            pltpu.SemaphoreType.DMA,
            pltpu.SemaphoreType.DMA,
            pltpu.SemaphoreType.DMA,
            pltpu.SemaphoreType.DMA,
            pltpu.SemaphoreType.DMA,
        ],
    )
    def gather_kernel(cb_hbm, idx_hbm, out_hbm, idx2d, rows_v,
                      isem, g0, g1, w0, w1):
        wid = lax.axis_index("s") * nc + lax.axis_index("c")
        base = wid * b_per_w
        gsem = (g0, g1)
        wsem = (w0, w1)
        cps = [pltpu.async_copy(idx_hbm.at[pl.ds(base + ci * chunk, chunk)],
                                idx2d.at[ci], isem)
               for ci in range(nchunk)]
        for cp in cps:
            cp.wait()
        gathers = [None] * nchunk
        pending_w = [None, None]
        gathers[0] = pltpu.async_copy(cb_hbm.at[idx2d.at[0]],
                                      rows_v.at[0], gsem[0])
        for ci in range(nchunk):
            b = ci % 2
            nb = (ci + 1) % 2
            if ci + 1 < nchunk:
                if pending_w[nb] is not None:
                    pending_w[nb].wait()
                    pending_w[nb] = None
                gathers[ci + 1] = pltpu.async_copy(
                    cb_hbm.at[idx2d.at[ci + 1]], rows_v.at[nb], gsem[nb])
            gathers[ci].wait()
            pending_w[b] = pltpu.async_copy(
                rows_v.at[b], out_hbm.at[pl.ds(base + ci * chunk, chunk)],
                wsem[b])
        for b in range(2):
            if pending_w[b] is not None:
                pending_w[b].wait()

    return gather_kernel


def _copy_block_kernel(q_ref, buf_ref, o_ref):
    o_ref[...] = q_ref[...]


@functools.lru_cache(maxsize=None)
def _make_assembler(n, ch, d, row0):
    nblkc = ch // _MBLK
    blk0 = row0 // _MBLK
    return pl.pallas_call(
        _copy_block_kernel,
        grid=(nblkc,),
        in_specs=[
            pl.BlockSpec((_MBLK, d), lambda i: (i, 0)),
            pl.BlockSpec(memory_space=pl.ANY),
        ],
        out_specs=pl.BlockSpec((_MBLK, d), lambda i: (blk0 + i, 0)),
        out_shape=jax.ShapeDtypeStruct((n, d), jnp.float32),
        input_output_aliases={1: 0},
    )


def kernel(in_feas, codebook):
    bq, lq, cq = in_feas.shape
    x = in_feas.reshape(-1, cq)
    n = x.shape[0]
    k, d = codebook.shape
    nblk = n // _MBLK
    blk_per_ch = nblk // _NCH
    ch = blk_per_ch * _MBLK
    n_sc = _NCH - 1
    idxs = []
    qparts = []
    for i in range(n_sc):
        scorer = _make_tc_scorer(n, cq, k, i * blk_per_ch, blk_per_ch)
        idx_i = scorer(x, codebook).reshape(-1)
        idxs.append(idx_i)
        nout = n if i == 0 else ch
        qparts.append(_make_sc_gather(ch, nout, k, d)(codebook, idx_i))
    quant = qparts[0]
    for i in range(n_sc, _NCH):
        scorer_q = _make_tc_scorer_quant(n, cq, k, i * blk_per_ch, blk_per_ch)
        idx_i, quant = scorer_q(x, codebook, quant)
        idxs.append(idx_i.reshape(-1))
    for i in range(1, n_sc):
        quant = lax.dynamic_update_slice(quant, qparts[i], (i * ch, 0))
    idx = jnp.concatenate(idxs, axis=0)
    h = int(math.sqrt(lq))
    w = lq // h
    return quant.reshape(bq, lq, cq), idx.reshape(bq, h, w)


# speedup vs baseline: 1.1006x; 1.0076x over previous
import functools
import math

import jax
import jax.numpy as jnp
from jax import lax
from jax.experimental import pallas as pl
from jax.experimental.pallas import tpu as pltpu
from jax.experimental.pallas import tpu_sc as plsc

_MBLK = 1152
_NCH = 4


def _dist_argmin_kernel(x_ref, cb_ref, idx_ref, esq_ref):
    cb = cb_ref[...]

    @pl.when(pl.program_id(0) == 0)
    def _():
        esq_ref[...] = jnp.sum(cb * cb, axis=1)[None, :]

    x = x_ref[...]
    x_sq = jnp.sum(x * x, axis=1, keepdims=True)
    e_sq = esq_ref[...]
    dot = lax.dot_general(x, cb, (((1,), (1,)), ((), ())),
                          preferred_element_type=jnp.float32)
    dist = x_sq + e_sq - 2.0 * dot
    idx_ref[0, 0, :] = jnp.argmin(dist, axis=1).astype(jnp.int32)


@functools.lru_cache(maxsize=None)
def _make_tc_scorer(n, c, k, blk0, nblk):
    return pl.pallas_call(
        _dist_argmin_kernel,
        grid=(nblk,),
        in_specs=[
            pl.BlockSpec((_MBLK, c), lambda i: (blk0 + i, 0)),
            pl.BlockSpec((k, c), lambda i: (0, 0)),
        ],
        out_specs=pl.BlockSpec((1, 1, _MBLK), lambda i: (i, 0, 0)),
        out_shape=jax.ShapeDtypeStruct((nblk, 1, _MBLK), jnp.int32),
        scratch_shapes=[pltpu.VMEM((1, k), jnp.float32)],
    )


def _dist_argmin_quant_kernel(x_ref, cb_ref, buf_ref, idx_ref, out_ref,
                              esq_ref, cbhi_ref, cblo_ref):
    mblk = x_ref.shape[0]
    kk = cb_ref.shape[0]
    cb = cb_ref[...]

    @pl.when(pl.program_id(0) == 0)
    def _():
        esq_ref[...] = jnp.sum(cb * cb, axis=1)[None, :]
        hi = cb.astype(jnp.bfloat16)
        cbhi_ref[...] = hi
        cblo_ref[...] = (cb - hi.astype(jnp.float32)).astype(jnp.bfloat16)

    x = x_ref[...]
    x_sq = jnp.sum(x * x, axis=1, keepdims=True)
    e_sq = esq_ref[...]
    dot = lax.dot_general(x, cb, (((1,), (1,)), ((), ())),
                          preferred_element_type=jnp.float32)
    dist = x_sq + e_sq - 2.0 * dot
    idx = jnp.argmin(dist, axis=1).astype(jnp.int32)
    idx_ref[0, 0, :] = idx
    iota = lax.broadcasted_iota(jnp.int32, (mblk, kk), 1)
    onehot = (iota == idx[:, None]).astype(jnp.bfloat16)
    q = lax.dot_general(onehot, cbhi_ref[...], (((1,), (0,)), ((), ())),
                        preferred_element_type=jnp.float32)
    q = q + lax.dot_general(onehot, cblo_ref[...], (((1,), (0,)), ((), ())),
                            preferred_element_type=jnp.float32)
    out_ref[...] = q


@functools.lru_cache(maxsize=None)
def _make_tc_scorer_quant(n, c, k, blk0, nblk):
    return pl.pallas_call(
        _dist_argmin_quant_kernel,
        grid=(nblk,),
        in_specs=[
            pl.BlockSpec((_MBLK, c), lambda i: (blk0 + i, 0)),
            pl.BlockSpec((k, c), lambda i: (0, 0)),
            pl.BlockSpec(memory_space=pl.ANY),
        ],
        out_specs=[
            pl.BlockSpec((1, 1, _MBLK), lambda i: (i, 0, 0)),
            pl.BlockSpec((_MBLK, c), lambda i: (blk0 + i, 0)),
        ],
        out_shape=[
            jax.ShapeDtypeStruct((nblk, 1, _MBLK), jnp.int32),
            jax.ShapeDtypeStruct((n, c), jnp.float32),
        ],
        scratch_shapes=[pltpu.VMEM((1, k), jnp.float32),
                        pltpu.VMEM((k, c), jnp.bfloat16),
                        pltpu.VMEM((k, c), jnp.bfloat16)],
        input_output_aliases={2: 1},
    )


@functools.lru_cache(maxsize=None)
def _make_sc_gather(nidx, nout, k, d):
    nc, ns = 2, 16
    nw = nc * ns
    assert nidx % nw == 0
    b_per_w = nidx // nw
    chunk = 96 if b_per_w % 96 == 0 else 72
    assert b_per_w % chunk == 0 and chunk % 8 == 0
    nchunk = b_per_w // chunk
    mesh = plsc.VectorSubcoreMesh(core_axis_name="c", subcore_axis_name="s")

    @functools.partial(
        pl.kernel,
        mesh=mesh,
        out_type=jax.ShapeDtypeStruct((nout, d), jnp.float32),
        scratch_types=[
            pltpu.VMEM((nchunk, chunk), jnp.int32),
            pltpu.VMEM((2, chunk, d), jnp.float32),
            pltpu.SemaphoreType.DMA,
            pltpu.SemaphoreType.DMA,
            pltpu.SemaphoreType.DMA,
            pltpu.SemaphoreType.DMA,
            pltpu.SemaphoreType.DMA,
        ],
    )
    def gather_kernel(cb_hbm, idx_hbm, out_hbm, idx2d, rows_v,
                      isem, g0, g1, w0, w1):
        wid = lax.axis_index("s") * nc + lax.axis_index("c")
        base = wid * b_per_w
        gsem = (g0, g1)
        wsem = (w0, w1)
        cps = [pltpu.async_copy(idx_hbm.at[pl.ds(base + ci * chunk, chunk)],
                                idx2d.at[ci], isem)
               for ci in range(nchunk)]
        for cp in cps:
            cp.wait()
        gathers = [None] * nchunk
        pending_w = [None, None]
        gathers[0] = pltpu.async_copy(cb_hbm.at[idx2d.at[0]],
                                      rows_v.at[0], gsem[0])
        for ci in range(nchunk):
            b = ci % 2
            nb = (ci + 1) % 2
            if ci + 1 < nchunk:
                if pending_w[nb] is not None:
                    pending_w[nb].wait()
                    pending_w[nb] = None
                gathers[ci + 1] = pltpu.async_copy(
                    cb_hbm.at[idx2d.at[ci + 1]], rows_v.at[nb], gsem[nb])
            gathers[ci].wait()
            pending_w[b] = pltpu.async_copy(
                rows_v.at[b], out_hbm.at[pl.ds(base + ci * chunk, chunk)],
                wsem[b])
        for b in range(2):
            if pending_w[b] is not None:
                pending_w[b].wait()

    return gather_kernel


def _copy_block_kernel(q_ref, buf_ref, o_ref):
    o_ref[...] = q_ref[...]


@functools.lru_cache(maxsize=None)
def _make_assembler(n, ch, d, row0):
    nblkc = ch // _MBLK
    blk0 = row0 // _MBLK
    return pl.pallas_call(
        _copy_block_kernel,
        grid=(nblkc,),
        in_specs=[
            pl.BlockSpec((_MBLK, d), lambda i: (i, 0)),
            pl.BlockSpec(memory_space=pl.ANY),
        ],
        out_specs=pl.BlockSpec((_MBLK, d), lambda i: (blk0 + i, 0)),
        out_shape=jax.ShapeDtypeStruct((n, d), jnp.float32),
        input_output_aliases={1: 0},
    )


def kernel(in_feas, codebook):
    bq, lq, cq = in_feas.shape
    x = in_feas.reshape(-1, cq)
    n = x.shape[0]
    k, d = codebook.shape
    nblk = n // _MBLK
    blk_per_ch = nblk // _NCH
    ch = blk_per_ch * _MBLK
    n_sc = _NCH - 1
    idxs = []
    qparts = []
    for i in range(n_sc):
        scorer = _make_tc_scorer(n, cq, k, i * blk_per_ch, blk_per_ch)
        idx_i = scorer(x, codebook).reshape(-1)
        idxs.append(idx_i)
        nout = n if i == 0 else ch
        qparts.append(_make_sc_gather(ch, nout, k, d)(codebook, idx_i))
    quant = qparts[0]
    for i in range(n_sc, _NCH):
        scorer_q = _make_tc_scorer_quant(n, cq, k, i * blk_per_ch, blk_per_ch)
        idx_i, quant = scorer_q(x, codebook, quant)
        idxs.append(idx_i.reshape(-1))
    for i in range(1, n_sc):
        quant = lax.dynamic_update_slice(quant, qparts[i], (i * ch, 0))
    idx = jnp.concatenate(idxs, axis=0)
    h = int(math.sqrt(lq))
    w = lq // h
    return quant.reshape(bq, lq, cq), idx.reshape(bq, h, w)


# speedup vs baseline: 1.1833x; 1.0752x over previous
import functools
import math

import jax
import jax.numpy as jnp
from jax import lax
from jax.experimental import pallas as pl
from jax.experimental.pallas import tpu as pltpu
from jax.experimental.pallas import tpu_sc as plsc

_MBLK = 1024
_NCH = 3


def _dist_argmin_kernel(x_ref, cb_ref, idx_ref, esq_ref):
    cb = cb_ref[...]

    @pl.when(pl.program_id(0) == 0)
    def _():
        esq_ref[...] = jnp.sum(cb * cb, axis=1)[None, :]

    x = x_ref[...]
    x_sq = jnp.sum(x * x, axis=1, keepdims=True)
    e_sq = esq_ref[...]
    dot = lax.dot_general(x, cb, (((1,), (1,)), ((), ())),
                          preferred_element_type=jnp.float32)
    dist = x_sq + e_sq - 2.0 * dot
    idx_ref[0, 0, :] = jnp.argmin(dist, axis=1).astype(jnp.int32)


@functools.lru_cache(maxsize=None)
def _make_tc_scorer(n, c, k, blk0, nblk):
    return pl.pallas_call(
        _dist_argmin_kernel,
        grid=(nblk,),
        in_specs=[
            pl.BlockSpec((_MBLK, c), lambda i: (blk0 + i, 0)),
            pl.BlockSpec((k, c), lambda i: (0, 0)),
        ],
        out_specs=pl.BlockSpec((1, 1, _MBLK), lambda i: (i, 0, 0)),
        out_shape=jax.ShapeDtypeStruct((nblk, 1, _MBLK), jnp.int32),
        scratch_shapes=[pltpu.VMEM((1, k), jnp.float32)],
    )


def _dist_argmin_quant_kernel(x_ref, cb_ref, buf_ref, idx_ref, out_ref,
                              esq_ref, cbhi_ref, cblo_ref):
    mblk = x_ref.shape[0]
    kk = cb_ref.shape[0]
    cb = cb_ref[...]

    @pl.when(pl.program_id(0) == 0)
    def _():
        esq_ref[...] = jnp.sum(cb * cb, axis=1)[None, :]
        hi = cb.astype(jnp.bfloat16)
        cbhi_ref[...] = hi
        cblo_ref[...] = (cb - hi.astype(jnp.float32)).astype(jnp.bfloat16)

    x = x_ref[...]
    x_sq = jnp.sum(x * x, axis=1, keepdims=True)
    e_sq = esq_ref[...]
    dot = lax.dot_general(x, cb, (((1,), (1,)), ((), ())),
                          preferred_element_type=jnp.float32)
    dist = x_sq + e_sq - 2.0 * dot
    idx = jnp.argmin(dist, axis=1).astype(jnp.int32)
    idx_ref[0, 0, :] = idx
    iota = lax.broadcasted_iota(jnp.int32, (mblk, kk), 1)
    onehot = (iota == idx[:, None]).astype(jnp.bfloat16)
    q = lax.dot_general(onehot, cbhi_ref[...], (((1,), (0,)), ((), ())),
                        preferred_element_type=jnp.float32)
    q = q + lax.dot_general(onehot, cblo_ref[...], (((1,), (0,)), ((), ())),
                            preferred_element_type=jnp.float32)
    out_ref[...] = q


@functools.lru_cache(maxsize=None)
def _make_tc_scorer_quant(n, c, k, blk0, nblk):
    return pl.pallas_call(
        _dist_argmin_quant_kernel,
        grid=(nblk,),
        in_specs=[
            pl.BlockSpec((_MBLK, c), lambda i: (blk0 + i, 0)),
            pl.BlockSpec((k, c), lambda i: (0, 0)),
            pl.BlockSpec(memory_space=pl.ANY),
        ],
        out_specs=[
            pl.BlockSpec((1, 1, _MBLK), lambda i: (i, 0, 0)),
            pl.BlockSpec((_MBLK, c), lambda i: (blk0 + i, 0)),
        ],
        out_shape=[
            jax.ShapeDtypeStruct((nblk, 1, _MBLK), jnp.int32),
            jax.ShapeDtypeStruct((n, c), jnp.float32),
        ],
        scratch_shapes=[pltpu.VMEM((1, k), jnp.float32),
                        pltpu.VMEM((k, c), jnp.bfloat16),
                        pltpu.VMEM((k, c), jnp.bfloat16)],
        input_output_aliases={2: 1},
    )


@functools.lru_cache(maxsize=None)
def _make_sc_gather(nidx, nout, k, d):
    nc, ns = 2, 16
    nw = nc * ns
    assert nidx % nw == 0
    b_per_w = nidx // nw
    chunk = 96 if b_per_w % 96 == 0 else 72
    assert b_per_w % chunk == 0 and chunk % 8 == 0
    nchunk = b_per_w // chunk
    mesh = plsc.VectorSubcoreMesh(core_axis_name="c", subcore_axis_name="s")

    @functools.partial(
        pl.kernel,
        mesh=mesh,
        out_type=jax.ShapeDtypeStruct((nout, d), jnp.float32),
        scratch_types=[
            pltpu.VMEM((nchunk, chunk), jnp.int32),
            pltpu.VMEM((2, chunk, d), jnp.float32),
            pltpu.SemaphoreType.DMA,
            pltpu.SemaphoreType.DMA,
            pltpu.SemaphoreType.DMA,
            pltpu.SemaphoreType.DMA,
            pltpu.SemaphoreType.DMA,
        ],
    )
    def gather_kernel(cb_hbm, idx_hbm, out_hbm, idx2d, rows_v,
                      isem, g0, g1, w0, w1):
        wid = lax.axis_index("s") * nc + lax.axis_index("c")
        base = wid * b_per_w
        gsem = (g0, g1)
        wsem = (w0, w1)
        cps = [pltpu.async_copy(idx_hbm.at[pl.ds(base + ci * chunk, chunk)],
                                idx2d.at[ci], isem)
               for ci in range(nchunk)]
        for cp in cps:
            cp.wait()
        gathers = [None] * nchunk
        pending_w = [None, None]
        gathers[0] = pltpu.async_copy(cb_hbm.at[idx2d.at[0]],
                                      rows_v.at[0], gsem[0])
        for ci in range(nchunk):
            b = ci % 2
            nb = (ci + 1) % 2
            if ci + 1 < nchunk:
                if pending_w[nb] is not None:
                    pending_w[nb].wait()
                    pending_w[nb] = None
                gathers[ci + 1] = pltpu.async_copy(
                    cb_hbm.at[idx2d.at[ci + 1]], rows_v.at[nb], gsem[nb])
            gathers[ci].wait()
            pending_w[b] = pltpu.async_copy(
                rows_v.at[b], out_hbm.at[pl.ds(base + ci * chunk, chunk)],
                wsem[b])
        for b in range(2):
            if pending_w[b] is not None:
                pending_w[b].wait()

    return gather_kernel


def _copy_block_kernel(q_ref, buf_ref, o_ref):
    o_ref[...] = q_ref[...]


@functools.lru_cache(maxsize=None)
def _make_assembler(n, ch, d, row0):
    nblkc = ch // _MBLK
    blk0 = row0 // _MBLK
    return pl.pallas_call(
        _copy_block_kernel,
        grid=(nblkc,),
        in_specs=[
            pl.BlockSpec((_MBLK, d), lambda i: (i, 0)),
            pl.BlockSpec(memory_space=pl.ANY),
        ],
        out_specs=pl.BlockSpec((_MBLK, d), lambda i: (blk0 + i, 0)),
        out_shape=jax.ShapeDtypeStruct((n, d), jnp.float32),
        input_output_aliases={1: 0},
    )


def kernel(in_feas, codebook):
    bq, lq, cq = in_feas.shape
    x = in_feas.reshape(-1, cq)
    n = x.shape[0]
    k, d = codebook.shape
    nblk = n // _MBLK
    blk_per_ch = nblk // _NCH
    ch = blk_per_ch * _MBLK
    n_sc = _NCH - 1
    idxs = []
    qparts = []
    for i in range(n_sc):
        scorer = _make_tc_scorer(n, cq, k, i * blk_per_ch, blk_per_ch)
        idx_i = scorer(x, codebook).reshape(-1)
        idxs.append(idx_i)
        nout = n if i == 0 else ch
        qparts.append(_make_sc_gather(ch, nout, k, d)(codebook, idx_i))
    quant = qparts[0]
    for i in range(n_sc, _NCH):
        scorer_q = _make_tc_scorer_quant(n, cq, k, i * blk_per_ch, blk_per_ch)
        idx_i, quant = scorer_q(x, codebook, quant)
        idxs.append(idx_i.reshape(-1))
    for i in range(1, n_sc):
        quant = lax.dynamic_update_slice(quant, qparts[i], (i * ch, 0))
    idx = jnp.concatenate(idxs, axis=0)
    h = int(math.sqrt(lq))
    w = lq // h
    return quant.reshape(bq, lq, cq), idx.reshape(bq, h, w)


# speedup vs baseline: 1.1948x; 1.0097x over previous
import functools
import math

import jax
import jax.numpy as jnp
from jax import lax
from jax.experimental import pallas as pl
from jax.experimental.pallas import tpu as pltpu
from jax.experimental.pallas import tpu_sc as plsc

_MBLK = 1536
_NCH = 3


def _dist_argmin_kernel(x_ref, cb_ref, idx_ref, esq_ref):
    cb = cb_ref[...]

    @pl.when(pl.program_id(0) == 0)
    def _():
        esq_ref[...] = jnp.sum(cb * cb, axis=1)[None, :]

    x = x_ref[...]
    x_sq = jnp.sum(x * x, axis=1, keepdims=True)
    e_sq = esq_ref[...]
    dot = lax.dot_general(x, cb, (((1,), (1,)), ((), ())),
                          preferred_element_type=jnp.float32)
    dist = x_sq + e_sq - 2.0 * dot
    idx_ref[0, 0, :] = jnp.argmin(dist, axis=1).astype(jnp.int32)


@functools.lru_cache(maxsize=None)
def _make_tc_scorer(n, c, k, blk0, nblk):
    return pl.pallas_call(
        _dist_argmin_kernel,
        grid=(nblk,),
        in_specs=[
            pl.BlockSpec((_MBLK, c), lambda i: (blk0 + i, 0)),
            pl.BlockSpec((k, c), lambda i: (0, 0)),
        ],
        out_specs=pl.BlockSpec((1, 1, _MBLK), lambda i: (i, 0, 0)),
        out_shape=jax.ShapeDtypeStruct((nblk, 1, _MBLK), jnp.int32),
        scratch_shapes=[pltpu.VMEM((1, k), jnp.float32)],
    )


def _dist_argmin_quant_kernel(x_ref, cb_ref, buf_ref, idx_ref, out_ref,
                              esq_ref, cbhi_ref, cblo_ref):
    mblk = x_ref.shape[0]
    kk = cb_ref.shape[0]
    cb = cb_ref[...]

    @pl.when(pl.program_id(0) == 0)
    def _():
        esq_ref[...] = jnp.sum(cb * cb, axis=1)[None, :]
        hi = cb.astype(jnp.bfloat16)
        cbhi_ref[...] = hi
        cblo_ref[...] = (cb - hi.astype(jnp.float32)).astype(jnp.bfloat16)

    x = x_ref[...]
    x_sq = jnp.sum(x * x, axis=1, keepdims=True)
    e_sq = esq_ref[...]
    dot = lax.dot_general(x, cb, (((1,), (1,)), ((), ())),
                          preferred_element_type=jnp.float32)
    dist = x_sq + e_sq - 2.0 * dot
    idx = jnp.argmin(dist, axis=1).astype(jnp.int32)
    idx_ref[0, 0, :] = idx
    iota = lax.broadcasted_iota(jnp.int32, (mblk, kk), 1)
    onehot = (iota == idx[:, None]).astype(jnp.bfloat16)
    q = lax.dot_general(onehot, cbhi_ref[...], (((1,), (0,)), ((), ())),
                        preferred_element_type=jnp.float32)
    q = q + lax.dot_general(onehot, cblo_ref[...], (((1,), (0,)), ((), ())),
                            preferred_element_type=jnp.float32)
    out_ref[...] = q


@functools.lru_cache(maxsize=None)
def _make_tc_scorer_quant(n, c, k, blk0, nblk):
    return pl.pallas_call(
        _dist_argmin_quant_kernel,
        grid=(nblk,),
        in_specs=[
            pl.BlockSpec((_MBLK, c), lambda i: (blk0 + i, 0)),
            pl.BlockSpec((k, c), lambda i: (0, 0)),
            pl.BlockSpec(memory_space=pl.ANY),
        ],
        out_specs=[
            pl.BlockSpec((1, 1, _MBLK), lambda i: (i, 0, 0)),
            pl.BlockSpec((_MBLK, c), lambda i: (blk0 + i, 0)),
        ],
        out_shape=[
            jax.ShapeDtypeStruct((nblk, 1, _MBLK), jnp.int32),
            jax.ShapeDtypeStruct((n, c), jnp.float32),
        ],
        scratch_shapes=[pltpu.VMEM((1, k), jnp.float32),
                        pltpu.VMEM((k, c), jnp.bfloat16),
                        pltpu.VMEM((k, c), jnp.bfloat16)],
        input_output_aliases={2: 1},
    )


@functools.lru_cache(maxsize=None)
def _make_sc_gather(nidx, nout, k, d):
    nc, ns = 2, 16
    nw = nc * ns
    assert nidx % nw == 0
    b_per_w = nidx // nw
    chunk = 96 if b_per_w % 96 == 0 else 72
    assert b_per_w % chunk == 0 and chunk % 8 == 0
    nchunk = b_per_w // chunk
    mesh = plsc.VectorSubcoreMesh(core_axis_name="c", subcore_axis_name="s")

    @functools.partial(
        pl.kernel,
        mesh=mesh,
        out_type=jax.ShapeDtypeStruct((nout, d), jnp.float32),
        scratch_types=[
            pltpu.VMEM((nchunk, chunk), jnp.int32),
            pltpu.VMEM((2, chunk, d), jnp.float32),
            pltpu.SemaphoreType.DMA,
            pltpu.SemaphoreType.DMA,
            pltpu.SemaphoreType.DMA,
            pltpu.SemaphoreType.DMA,
            pltpu.SemaphoreType.DMA,
        ],
    )
    def gather_kernel(cb_hbm, idx_hbm, out_hbm, idx2d, rows_v,
                      isem, g0, g1, w0, w1):
        wid = lax.axis_index("s") * nc + lax.axis_index("c")
        base = wid * b_per_w
        gsem = (g0, g1)
        wsem = (w0, w1)
        cps = [pltpu.async_copy(idx_hbm.at[pl.ds(base + ci * chunk, chunk)],
                                idx2d.at[ci], isem)
               for ci in range(nchunk)]
        for cp in cps:
            cp.wait()
        gathers = [None] * nchunk
        pending_w = [None, None]
        gathers[0] = pltpu.async_copy(cb_hbm.at[idx2d.at[0]],
                                      rows_v.at[0], gsem[0])
        for ci in range(nchunk):
            b = ci % 2
            nb = (ci + 1) % 2
            if ci + 1 < nchunk:
                if pending_w[nb] is not None:
                    pending_w[nb].wait()
                    pending_w[nb] = None
                gathers[ci + 1] = pltpu.async_copy(
                    cb_hbm.at[idx2d.at[ci + 1]], rows_v.at[nb], gsem[nb])
            gathers[ci].wait()
            pending_w[b] = pltpu.async_copy(
                rows_v.at[b], out_hbm.at[pl.ds(base + ci * chunk, chunk)],
                wsem[b])
        for b in range(2):
            if pending_w[b] is not None:
                pending_w[b].wait()

    return gather_kernel


def _copy_block_kernel(q_ref, buf_ref, o_ref):
    o_ref[...] = q_ref[...]


@functools.lru_cache(maxsize=None)
def _make_assembler(n, ch, d, row0):
    nblkc = ch // _MBLK
    blk0 = row0 // _MBLK
    return pl.pallas_call(
        _copy_block_kernel,
        grid=(nblkc,),
        in_specs=[
            pl.BlockSpec((_MBLK, d), lambda i: (i, 0)),
            pl.BlockSpec(memory_space=pl.ANY),
        ],
        out_specs=pl.BlockSpec((_MBLK, d), lambda i: (blk0 + i, 0)),
        out_shape=jax.ShapeDtypeStruct((n, d), jnp.float32),
        input_output_aliases={1: 0},
    )


def kernel(in_feas, codebook):
    bq, lq, cq = in_feas.shape
    x = in_feas.reshape(-1, cq)
    n = x.shape[0]
    k, d = codebook.shape
    nblk = n // _MBLK
    blk_per_ch = nblk // _NCH
    ch = blk_per_ch * _MBLK
    n_sc = _NCH - 1
    idxs = []
    qparts = []
    for i in range(n_sc):
        scorer = _make_tc_scorer(n, cq, k, i * blk_per_ch, blk_per_ch)
        idx_i = scorer(x, codebook).reshape(-1)
        idxs.append(idx_i)
        nout = n if i == 0 else ch
        qparts.append(_make_sc_gather(ch, nout, k, d)(codebook, idx_i))
    quant = qparts[0]
    for i in range(n_sc, _NCH):
        scorer_q = _make_tc_scorer_quant(n, cq, k, i * blk_per_ch, blk_per_ch)
        idx_i, quant = scorer_q(x, codebook, quant)
        idxs.append(idx_i.reshape(-1))
    for i in range(1, n_sc):
        quant = lax.dynamic_update_slice(quant, qparts[i], (i * ch, 0))
    idx = jnp.concatenate(idxs, axis=0)
    h = int(math.sqrt(lq))
    w = lq // h
    return quant.reshape(bq, lq, cq), idx.reshape(bq, h, w)
